# Initial kernel scaffold; baseline (speedup 1.0000x reference)
import jax
import jax.numpy as jnp
from jax.experimental import pallas as pl


def kernel(features, W1, Om1, W2, Om2, W3, Om3, V0_w, V0_b, V1_w, V1_b, edge_index, batch):
    raise NotImplementedError("write your pallas kernel here")


# speedup vs baseline: 3.1785x; 3.1785x over previous
import functools

import jax
import jax.numpy as jnp
from jax import lax
from jax.experimental import pallas as pl
from jax.experimental.pallas import tpu as pltpu
from jax.experimental.pallas import tpu_sc as plsc

_N = 10000
_E = 320000
_D = 128
_NCLASS = 16
_NGRAPH = 128
_KAPPA = 0.9
_NITER = 25

_NPAD = 10240
_CHUNK = 128
_NTILES = 16
_NSC = 2
_NW = _NSC * _NTILES
_RPT = _NPAD // _NTILES

_NC_EDGE = 80
_EPAD = _NW * _CHUNK * _NC_EDGE
_NC_POOL = 8
_POOLPAD = _NW * _CHUNK * _NC_POOL


def _make_spmm(nchunks, nstages):
    assert nchunks % nstages == 0
    nch = nchunks // nstages
    assert nch >= 4 and nch % 2 == 0
    mesh = plsc.VectorSubcoreMesh(core_axis_name="c", subcore_axis_name="s")

    @functools.partial(
        pl.kernel,
        out_type=jax.ShapeDtypeStruct((_NSC, _NPAD, _D), jnp.float32),
        mesh=mesh,
        scratch_types=[
            pltpu.VMEM((nch, _CHUNK), jnp.int32),
            pltpu.VMEM((nch, _CHUNK), jnp.int32),
            pltpu.VMEM((_CHUNK, _D), jnp.float32),
            pltpu.VMEM((_CHUNK, _D), jnp.float32),
            pltpu.VMEM_SHARED((_NPAD, _D), jnp.float32),
            pltpu.SemaphoreType.DMA,
            pltpu.SemaphoreType.DMA,
        ],
    )
    def spmm(src_hbm, dst_hbm, s_hbm, out_hbm,
             src_v, dst_v, rows0, rows1, acc, sem0, sem1):
        c = lax.axis_index("c")
        s = lax.axis_index("s")
        gid = c * _NTILES + s

        zv = jnp.zeros((16,), jnp.float32)

        def zb(i, carry):
            for j in range(8):
                rows0[i, pl.ds(j * 16, 16)] = zv
            return carry

        lax.fori_loop(0, _CHUNK, zb, 0)
        for j in range(_RPT // _CHUNK):
            pltpu.sync_copy(rows0, acc.at[pl.ds(s * _RPT + j * _CHUNK, _CHUNK)])
        plsc.subcore_barrier()

        for h in range(nstages):
            base = gid * nchunks + h * nch
            pltpu.sync_copy(src_hbm.at[pl.ds(base, nch)], src_v)
            pltpu.sync_copy(dst_hbm.at[pl.ds(base, nch)], dst_v)

            pltpu.async_copy(s_hbm.at[src_v.at[0]], rows0, sem0)
            pltpu.async_copy(s_hbm.at[src_v.at[1]], rows1, sem1)

            def pair(i, carry):
                k0 = 2 * i
                k1 = k0 + 1
                pltpu.make_async_copy(s_hbm.at[src_v.at[k0]], rows0, sem0).wait()
                pltpu.sync_copy(rows0, acc.at[dst_v.at[k0]], add=True)

                @pl.when(k0 + 2 < nch)
                def _():
                    pltpu.async_copy(s_hbm.at[src_v.at[k0 + 2]], rows0, sem0)

                pltpu.make_async_copy(s_hbm.at[src_v.at[k1]], rows1, sem1).wait()
                pltpu.sync_copy(rows1, acc.at[dst_v.at[k1]], add=True)

                @pl.when(k1 + 2 < nch)
                def _():
                    pltpu.async_copy(s_hbm.at[src_v.at[k1 + 2]], rows1, sem1)

                return carry

            lax.fori_loop(0, nch // 2, pair, 0)

        plsc.subcore_barrier()
        pltpu.sync_copy(acc.at[pl.ds(s * _RPT, _RPT)],
                        out_hbm.at[c, pl.ds(s * _RPT, _RPT)])

    return spmm


_make_spmm = functools.lru_cache(maxsize=None)(_make_spmm)


def _spmm_edges(srcp, dstp, s):
    return _make_spmm(_NC_EDGE, 2)(srcp, dstp, s)


def _spmm_pool(srcp, dstp, s):
    return _make_spmm(_NC_POOL, 1)(srcp, dstp, s)


_RB = 1024


def _mm_pre(u, omt):
    def body(u_ref, w_ref, o_ref):
        o_ref[...] = jnp.dot(u_ref[...], w_ref[...],
                             preferred_element_type=jnp.float32)

    return pl.pallas_call(
        body,
        grid=(_NPAD // _RB,),
        in_specs=[pl.BlockSpec((_RB, _D), lambda i: (i, 0)),
                  pl.BlockSpec((_D, _D), lambda i: (0, 0))],
        out_specs=pl.BlockSpec((_RB, _D), lambda i: (i, 0)),
        out_shape=jax.ShapeDtypeStruct((_NPAD, _D), jnp.float32),
    )(u, omt)


def _bsum(p):
    def body(p_ref, b_ref, x_ref):
        b = p_ref[0] + p_ref[1]
        b_ref[...] = b
        x_ref[...] = jnp.maximum(b, 0.0)

    return pl.pallas_call(
        body,
        grid=(_NPAD // _RB,),
        in_specs=[pl.BlockSpec((2, _RB, _D), lambda i: (0, i, 0))],
        out_specs=[pl.BlockSpec((_RB, _D), lambda i: (i, 0)),
                   pl.BlockSpec((_RB, _D), lambda i: (i, 0))],
        out_shape=[jax.ShapeDtypeStruct((_NPAD, _D), jnp.float32),
                   jax.ShapeDtypeStruct((_NPAD, _D), jnp.float32)],
    )(p)


def _iter_step(g, wpt, b):
    def body(g_ref, w_ref, b_ref, x_ref):
        y = jnp.dot(g_ref[0] + g_ref[1], w_ref[...],
                    preferred_element_type=jnp.float32)
        x_ref[...] = jnp.maximum(y + b_ref[...], 0.0)

    return pl.pallas_call(
        body,
        grid=(_NPAD // _RB,),
        in_specs=[pl.BlockSpec((2, _RB, _D), lambda i: (0, i, 0)),
                  pl.BlockSpec((_D, _D), lambda i: (0, 0)),
                  pl.BlockSpec((_RB, _D), lambda i: (i, 0))],
        out_specs=pl.BlockSpec((_RB, _D), lambda i: (i, 0)),
        out_shape=jax.ShapeDtypeStruct((_NPAD, _D), jnp.float32),
    )(g, wpt, b)


def _head(q, v0wt, v0b, v1wt, v1b):
    def body(q_ref, w0_ref, b0_ref, w1_ref, b1_ref, o_ref):
        pooled = q_ref[0] + q_ref[1]
        h = jnp.dot(pooled, w0_ref[...], preferred_element_type=jnp.float32)
        h = jnp.maximum(h + b0_ref[...], 0.0)
        lg = jnp.dot(h, w1_ref[...], preferred_element_type=jnp.float32)
        lg = lg + b1_ref[...]
        m = jnp.max(lg, axis=1, keepdims=True)
        lse = jnp.log(jnp.sum(jnp.exp(lg - m), axis=1, keepdims=True))
        o_ref[...] = lg - m - lse

    return pl.pallas_call(
        body,
        out_shape=jax.ShapeDtypeStruct((_NGRAPH, _NCLASS), jnp.float32),
    )(q, v0wt, v0b, v1wt, v1b)


def _project_l1_rows(w, v):
    a = jnp.abs(w)
    srt = jnp.sort(a, axis=1)[:, ::-1]
    cssv = jnp.cumsum(srt, axis=1) - v
    ind = jnp.arange(1, w.shape[1] + 1, dtype=w.dtype)
    cond = (srt - cssv / ind) > 0
    rho = jnp.maximum(jnp.sum(cond, axis=1), 1)
    theta = jnp.take_along_axis(cssv, (rho - 1)[:, None], axis=1)[:, 0]
    theta = jnp.maximum(theta / rho.astype(w.dtype), 0.0)
    proj = jnp.sign(w) * jnp.maximum(a - theta[:, None], 0.0)
    needs = a.sum(axis=1) > v
    return jnp.where(needs[:, None], proj, w)


def _pad_indices(idx, total, fill=None):
    idx = idx.astype(jnp.int32)
    npad = total - idx.shape[0]
    if fill is None:
        pad = (jnp.arange(npad, dtype=jnp.int32) * 8) % _NPAD
    else:
        pad = jnp.full((npad,), fill, jnp.int32)
    return jnp.concatenate([idx, pad]).reshape(-1, _CHUNK)


def kernel(features, W1, Om1, W2, Om2, W3, Om3,
           V0_w, V0_b, V1_w, V1_b, edge_index, batch):
    srcp = _pad_indices(edge_index[0], _EPAD, _NPAD - 1)
    dstp = _pad_indices(edge_index[1], _EPAD)
    pool_src = _pad_indices(jnp.arange(_N, dtype=jnp.int32), _POOLPAD, _NPAD - 1)
    pool_dst = _pad_indices(batch, _POOLPAD)

    u = jnp.pad(features.T, ((0, _NPAD - _N), (0, 0)))

    for w_raw, om in ((W1, Om1), (W2, Om2), (W3, Om3)):
        wpt = _project_l1_rows(w_raw, _KAPPA).T
        t1 = _mm_pre(u, om.T)
        p = _spmm_edges(srcp, dstp, t1)
        bmat, x = _bsum(p)
        for _ in range(_NITER - 1):
            g = _spmm_edges(srcp, dstp, x)
            x = _iter_step(g, wpt, bmat)
        u = x

    q = _spmm_pool(pool_src, pool_dst, u)
    return _head(q[:, :_NGRAPH, :], V0_w.T, V0_b.reshape(1, -1),
                 V1_w.T, V1_b.reshape(1, -1))


# speedup vs baseline: 3.6063x; 1.1346x over previous
import functools

import jax
import jax.numpy as jnp
from jax import lax
from jax.experimental import pallas as pl
from jax.experimental.pallas import tpu as pltpu
from jax.experimental.pallas import tpu_sc as plsc

_N = 10000
_E = 320000
_D = 128
_NCLASS = 16
_NGRAPH = 128
_KAPPA = 0.9
_NITER = 25

_NPAD = 10240
_CHUNK = 128
_NTILES = 16
_NSC = 2
_NW = _NSC * _NTILES
_RPT = _NPAD // _NTILES

_NC_EDGE = 80
_EPAD = _NW * _CHUNK * _NC_EDGE
_NC_POOL = 8
_POOLPAD = _NW * _CHUNK * _NC_POOL


def _make_spmm(nchunks, nstages):
    assert nchunks % nstages == 0
    nch = nchunks // nstages
    assert nch >= 4 and nch % 2 == 0
    mesh = plsc.VectorSubcoreMesh(core_axis_name="c", subcore_axis_name="s")

    @functools.partial(
        pl.kernel,
        out_type=jax.ShapeDtypeStruct((_NSC, _NPAD, _D), jnp.float32),
        mesh=mesh,
        scratch_types=[
            pltpu.VMEM((nch, _CHUNK), jnp.int32),
            pltpu.VMEM((nch, _CHUNK), jnp.int32),
            pltpu.VMEM((_CHUNK, _D), jnp.float32),
            pltpu.VMEM((_CHUNK, _D), jnp.float32),
            pltpu.VMEM_SHARED((_NPAD, _D), jnp.float32),
            pltpu.SemaphoreType.DMA,
            pltpu.SemaphoreType.DMA,
        ],
        compiler_params=pltpu.CompilerParams(use_tc_tiling_on_sc=False),
    )
    def spmm(src_hbm, dst_hbm, s_hbm, out_hbm,
             src_v, dst_v, rows0, rows1, acc, sem0, sem1):
        c = lax.axis_index("c")
        s = lax.axis_index("s")
        gid = c * _NTILES + s

        zv = jnp.zeros((16,), jnp.float32)

        def zb(i, carry):
            for j in range(8):
                rows0[i, pl.ds(j * 16, 16)] = zv
            return carry

        lax.fori_loop(0, _CHUNK, zb, 0)
        for j in range(_RPT // _CHUNK):
            pltpu.sync_copy(rows0, acc.at[pl.ds(s * _RPT + j * _CHUNK, _CHUNK)])
        plsc.subcore_barrier()

        for h in range(nstages):
            base = gid * nchunks + h * nch
            pltpu.sync_copy(src_hbm.at[pl.ds(base, nch)], src_v)
            pltpu.sync_copy(dst_hbm.at[pl.ds(base, nch)], dst_v)

            pltpu.async_copy(s_hbm.at[src_v.at[0]], rows0, sem0)
            pltpu.async_copy(s_hbm.at[src_v.at[1]], rows1, sem1)

            def pair(i, carry):
                k0 = 2 * i
                k1 = k0 + 1
                pltpu.make_async_copy(s_hbm.at[src_v.at[k0]], rows0, sem0).wait()
                pltpu.sync_copy(rows0, acc.at[dst_v.at[k0]], add=True)

                @pl.when(k0 + 2 < nch)
                def _():
                    pltpu.async_copy(s_hbm.at[src_v.at[k0 + 2]], rows0, sem0)

                pltpu.make_async_copy(s_hbm.at[src_v.at[k1]], rows1, sem1).wait()
                pltpu.sync_copy(rows1, acc.at[dst_v.at[k1]], add=True)

                @pl.when(k1 + 2 < nch)
                def _():
                    pltpu.async_copy(s_hbm.at[src_v.at[k1 + 2]], rows1, sem1)

                return carry

            lax.fori_loop(0, nch // 2, pair, 0)

        plsc.subcore_barrier()
        pltpu.sync_copy(acc.at[pl.ds(s * _RPT, _RPT)],
                        out_hbm.at[c, pl.ds(s * _RPT, _RPT)])

    return spmm


_make_spmm = functools.lru_cache(maxsize=None)(_make_spmm)


def _spmm_edges(srcp, dstp, s):
    return _make_spmm(_NC_EDGE, 2)(srcp, dstp, s)


def _spmm_pool(srcp, dstp, s):
    return _make_spmm(_NC_POOL, 1)(srcp, dstp, s)


_RB = 1024


def _mm_pre(u, omt):
    def body(u_ref, w_ref, o_ref):
        o_ref[...] = jnp.dot(u_ref[...], w_ref[...],
                             preferred_element_type=jnp.float32)

    return pl.pallas_call(
        body,
        grid=(_NPAD // _RB,),
        in_specs=[pl.BlockSpec((_RB, _D), lambda i: (i, 0)),
                  pl.BlockSpec((_D, _D), lambda i: (0, 0))],
        out_specs=pl.BlockSpec((_RB, _D), lambda i: (i, 0)),
        out_shape=jax.ShapeDtypeStruct((_NPAD, _D), jnp.float32),
    )(u, omt)


def _bsum(p):
    def body(p_ref, b_ref, x_ref):
        b = p_ref[0] + p_ref[1]
        b_ref[...] = b
        x_ref[...] = jnp.maximum(b, 0.0)

    return pl.pallas_call(
        body,
        grid=(_NPAD // _RB,),
        in_specs=[pl.BlockSpec((2, _RB, _D), lambda i: (0, i, 0))],
        out_specs=[pl.BlockSpec((_RB, _D), lambda i: (i, 0)),
                   pl.BlockSpec((_RB, _D), lambda i: (i, 0))],
        out_shape=[jax.ShapeDtypeStruct((_NPAD, _D), jnp.float32),
                   jax.ShapeDtypeStruct((_NPAD, _D), jnp.float32)],
    )(p)


def _iter_step(g, wpt, b):
    def body(g_ref, w_ref, b_ref, x_ref):
        y = jnp.dot(g_ref[0] + g_ref[1], w_ref[...],
                    preferred_element_type=jnp.float32)
        x_ref[...] = jnp.maximum(y + b_ref[...], 0.0)

    return pl.pallas_call(
        body,
        grid=(_NPAD // _RB,),
        in_specs=[pl.BlockSpec((2, _RB, _D), lambda i: (0, i, 0)),
                  pl.BlockSpec((_D, _D), lambda i: (0, 0)),
                  pl.BlockSpec((_RB, _D), lambda i: (i, 0))],
        out_specs=pl.BlockSpec((_RB, _D), lambda i: (i, 0)),
        out_shape=jax.ShapeDtypeStruct((_NPAD, _D), jnp.float32),
    )(g, wpt, b)


def _head(q, v0wt, v0b, v1wt, v1b):
    def body(q_ref, w0_ref, b0_ref, w1_ref, b1_ref, o_ref):
        pooled = q_ref[0] + q_ref[1]
        h = jnp.dot(pooled, w0_ref[...], preferred_element_type=jnp.float32)
        h = jnp.maximum(h + b0_ref[...], 0.0)
        lg = jnp.dot(h, w1_ref[...], preferred_element_type=jnp.float32)
        lg = lg + b1_ref[...]
        m = jnp.max(lg, axis=1, keepdims=True)
        lse = jnp.log(jnp.sum(jnp.exp(lg - m), axis=1, keepdims=True))
        o_ref[...] = lg - m - lse

    return pl.pallas_call(
        body,
        out_shape=jax.ShapeDtypeStruct((_NGRAPH, _NCLASS), jnp.float32),
    )(q, v0wt, v0b, v1wt, v1b)


def _project_l1_rows(w, v):
    a = jnp.abs(w)
    srt = jnp.sort(a, axis=1)[:, ::-1]
    cssv = jnp.cumsum(srt, axis=1) - v
    ind = jnp.arange(1, w.shape[1] + 1, dtype=w.dtype)
    cond = (srt - cssv / ind) > 0
    rho = jnp.maximum(jnp.sum(cond, axis=1), 1)
    theta = jnp.take_along_axis(cssv, (rho - 1)[:, None], axis=1)[:, 0]
    theta = jnp.maximum(theta / rho.astype(w.dtype), 0.0)
    proj = jnp.sign(w) * jnp.maximum(a - theta[:, None], 0.0)
    needs = a.sum(axis=1) > v
    return jnp.where(needs[:, None], proj, w)


def _pad_indices(idx, total, fill=None):
    idx = idx.astype(jnp.int32)
    npad = total - idx.shape[0]
    if fill is None:
        pad = (jnp.arange(npad, dtype=jnp.int32) * 8) % _NPAD
    else:
        pad = jnp.full((npad,), fill, jnp.int32)
    return jnp.concatenate([idx, pad]).reshape(-1, _CHUNK)


def kernel(features, W1, Om1, W2, Om2, W3, Om3,
           V0_w, V0_b, V1_w, V1_b, edge_index, batch):
    srcp = _pad_indices(edge_index[0], _EPAD, _NPAD - 1)
    dstp = _pad_indices(edge_index[1], _EPAD)
    pool_src = _pad_indices(jnp.arange(_N, dtype=jnp.int32), _POOLPAD, _NPAD - 1)
    pool_dst = _pad_indices(batch, _POOLPAD)

    u = jnp.pad(features.T, ((0, _NPAD - _N), (0, 0)))

    for w_raw, om in ((W1, Om1), (W2, Om2), (W3, Om3)):
        wpt = _project_l1_rows(w_raw, _KAPPA).T
        t1 = _mm_pre(u, om.T)
        p = _spmm_edges(srcp, dstp, t1)
        bmat, x = _bsum(p)
        for _ in range(_NITER - 1):
            g = _spmm_edges(srcp, dstp, x)
            x = _iter_step(g, wpt, bmat)
        u = x

    q = _spmm_pool(pool_src, pool_dst, u)
    return _head(q[:, :_NGRAPH, :], V0_w.T, V0_b.reshape(1, -1),
                 V1_w.T, V1_b.reshape(1, -1))


# speedup vs baseline: 8.2258x; 2.2810x over previous
import functools

import jax
import jax.numpy as jnp
from jax import lax
from jax.experimental import pallas as pl
from jax.experimental.pallas import tpu as pltpu
from jax.experimental.pallas import tpu_sc as plsc

_N = 10000
_E = 320000
_D = 128
_HD = 64
_NCLASS = 16
_NGRAPH = 128
_KAPPA = 0.9
_NITER = 25

_NPAD = 10240
_CHUNK = 128
_NTILES = 16
_NSC = 2
_RPT = _NPAD // _NTILES

_EC_TOTAL = 2560
_EPAD = _EC_TOTAL * _CHUNK
_PC_TOTAL = 256
_POOLPAD = _PC_TOTAL * _CHUNK


def _make_spmm(cpt, nstages):
    assert cpt % nstages == 0
    nch = cpt // nstages
    assert nch >= 4 and nch % 2 == 0
    mesh = plsc.VectorSubcoreMesh(core_axis_name="c", subcore_axis_name="s")

    @functools.partial(
        pl.kernel,
        out_type=jax.ShapeDtypeStruct((_NSC, _NPAD, _HD), jnp.float32),
        mesh=mesh,
        scratch_types=[
            pltpu.VMEM((nch, _CHUNK), jnp.int32),
            pltpu.VMEM((nch, _CHUNK), jnp.int32),
            pltpu.VMEM((_CHUNK, _HD), jnp.float32),
            pltpu.VMEM((_CHUNK, _HD), jnp.float32),
            pltpu.VMEM_SHARED((_NPAD, _HD), jnp.float32),
            pltpu.VMEM_SHARED((_NPAD, _HD), jnp.float32),
            pltpu.SemaphoreType.DMA,
            pltpu.SemaphoreType.DMA,
        ],
        compiler_params=pltpu.CompilerParams(use_tc_tiling_on_sc=False),
    )
    def spmm(src_hbm, dst_hbm, s_hbm, out_hbm,
             src_v, dst_v, rows0, rows1, s_sp, acc, sem0, sem1):
        c = lax.axis_index("c")
        s = lax.axis_index("s")

        pltpu.sync_copy(s_hbm.at[c, pl.ds(s * _RPT, _RPT)],
                        s_sp.at[pl.ds(s * _RPT, _RPT)])

        zv = jnp.zeros((16,), jnp.float32)

        def zb(i, carry):
            for j in range(_HD // 16):
                rows0[i, pl.ds(j * 16, 16)] = zv
            return carry

        lax.fori_loop(0, _CHUNK, zb, 0)
        for j in range(_RPT // _CHUNK):
            pltpu.sync_copy(rows0, acc.at[pl.ds(s * _RPT + j * _CHUNK, _CHUNK)])
        plsc.subcore_barrier()

        for h in range(nstages):
            base = s * cpt + h * nch
            pltpu.sync_copy(src_hbm.at[pl.ds(base, nch)], src_v)
            pltpu.sync_copy(dst_hbm.at[pl.ds(base, nch)], dst_v)

            pltpu.async_copy(s_sp.at[src_v.at[0]], rows0, sem0)
            pltpu.async_copy(s_sp.at[src_v.at[1]], rows1, sem1)

            def pair(i, carry):
                k0 = 2 * i
                k1 = k0 + 1
                pltpu.make_async_copy(s_sp.at[src_v.at[k0]], rows0, sem0).wait()
                pltpu.sync_copy(rows0, acc.at[dst_v.at[k0]], add=True)

                @pl.when(k0 + 2 < nch)
                def _():
                    pltpu.async_copy(s_sp.at[src_v.at[k0 + 2]], rows0, sem0)

                pltpu.make_async_copy(s_sp.at[src_v.at[k1]], rows1, sem1).wait()
                pltpu.sync_copy(rows1, acc.at[dst_v.at[k1]], add=True)

                @pl.when(k1 + 2 < nch)
                def _():
                    pltpu.async_copy(s_sp.at[src_v.at[k1 + 2]], rows1, sem1)

                return carry

            lax.fori_loop(0, nch // 2, pair, 0)

        plsc.subcore_barrier()
        pltpu.sync_copy(acc.at[pl.ds(s * _RPT, _RPT)],
                        out_hbm.at[c, pl.ds(s * _RPT, _RPT)])

    return spmm


_make_spmm = functools.lru_cache(maxsize=None)(_make_spmm)


def _spmm_edges(srcp, dstp, s):
    return _make_spmm(_EC_TOTAL // _NTILES, 4)(srcp, dstp, s)


def _spmm_pool(srcp, dstp, s):
    return _make_spmm(_PC_TOTAL // _NTILES, 1)(srcp, dstp, s)


_RB = 1024


def _mm_pre(u, omt):
    def body(u_ref, w_ref, o_ref):
        x = jnp.concatenate([u_ref[0], u_ref[1]], axis=1)
        y = jnp.dot(x, w_ref[...], preferred_element_type=jnp.float32)
        o_ref[0] = y[:, :_HD]
        o_ref[1] = y[:, _HD:]

    return pl.pallas_call(
        body,
        grid=(_NPAD // _RB,),
        in_specs=[pl.BlockSpec((2, _RB, _HD), lambda i: (0, i, 0)),
                  pl.BlockSpec((_D, _D), lambda i: (0, 0))],
        out_specs=pl.BlockSpec((2, _RB, _HD), lambda i: (0, i, 0)),
        out_shape=jax.ShapeDtypeStruct((2, _NPAD, _HD), jnp.float32),
    )(u, omt)


def _relu_half(b):
    def body(b_ref, x_ref):
        x_ref[...] = jnp.maximum(b_ref[...], 0.0)

    return pl.pallas_call(
        body,
        grid=(_NPAD // _RB,),
        in_specs=[pl.BlockSpec((2, _RB, _HD), lambda i: (0, i, 0))],
        out_specs=pl.BlockSpec((2, _RB, _HD), lambda i: (0, i, 0)),
        out_shape=jax.ShapeDtypeStruct((2, _NPAD, _HD), jnp.float32),
    )(b)


def _iter_step(g, wpt, b):
    def body(g_ref, w_ref, b_ref, x_ref):
        x = jnp.concatenate([g_ref[0], g_ref[1]], axis=1)
        y = jnp.dot(x, w_ref[...], preferred_element_type=jnp.float32)
        y = y + jnp.concatenate([b_ref[0], b_ref[1]], axis=1)
        y = jnp.maximum(y, 0.0)
        x_ref[0] = y[:, :_HD]
        x_ref[1] = y[:, _HD:]

    return pl.pallas_call(
        body,
        grid=(_NPAD // _RB,),
        in_specs=[pl.BlockSpec((2, _RB, _HD), lambda i: (0, i, 0)),
                  pl.BlockSpec((_D, _D), lambda i: (0, 0)),
                  pl.BlockSpec((2, _RB, _HD), lambda i: (0, i, 0))],
        out_specs=pl.BlockSpec((2, _RB, _HD), lambda i: (0, i, 0)),
        out_shape=jax.ShapeDtypeStruct((2, _NPAD, _HD), jnp.float32),
    )(g, wpt, b)


def _head(q, v0wt, v0b, v1wt, v1b):
    def body(q_ref, w0_ref, b0_ref, w1_ref, b1_ref, o_ref):
        pooled = jnp.concatenate([q_ref[0], q_ref[1]], axis=1)
        h = jnp.dot(pooled, w0_ref[...], preferred_element_type=jnp.float32)
        h = jnp.maximum(h + b0_ref[...], 0.0)
        lg = jnp.dot(h, w1_ref[...], preferred_element_type=jnp.float32)
        lg = lg + b1_ref[...]
        m = jnp.max(lg, axis=1, keepdims=True)
        lse = jnp.log(jnp.sum(jnp.exp(lg - m), axis=1, keepdims=True))
        o_ref[...] = lg - m - lse

    return pl.pallas_call(
        body,
        out_shape=jax.ShapeDtypeStruct((_NGRAPH, _NCLASS), jnp.float32),
    )(q, v0wt, v0b, v1wt, v1b)


def _project_l1_rows(w, v):
    a = jnp.abs(w)
    srt = jnp.sort(a, axis=1)[:, ::-1]
    cssv = jnp.cumsum(srt, axis=1) - v
    ind = jnp.arange(1, w.shape[1] + 1, dtype=w.dtype)
    cond = (srt - cssv / ind) > 0
    rho = jnp.maximum(jnp.sum(cond, axis=1), 1)
    theta = jnp.take_along_axis(cssv, (rho - 1)[:, None], axis=1)[:, 0]
    theta = jnp.maximum(theta / rho.astype(w.dtype), 0.0)
    proj = jnp.sign(w) * jnp.maximum(a - theta[:, None], 0.0)
    needs = a.sum(axis=1) > v
    return jnp.where(needs[:, None], proj, w)


def _pad_indices(idx, total, fill=None):
    idx = idx.astype(jnp.int32)
    npad = total - idx.shape[0]
    if fill is None:
        pad = (jnp.arange(npad, dtype=jnp.int32) * 8) % _NPAD
    else:
        pad = jnp.full((npad,), fill, jnp.int32)
    return jnp.concatenate([idx, pad]).reshape(-1, _CHUNK)


def kernel(features, W1, Om1, W2, Om2, W3, Om3,
           V0_w, V0_b, V1_w, V1_b, edge_index, batch):
    srcp = _pad_indices(edge_index[0], _EPAD, _NPAD - 1)
    dstp = _pad_indices(edge_index[1], _EPAD)
    pool_src = _pad_indices(jnp.arange(_N, dtype=jnp.int32), _POOLPAD, _NPAD - 1)
    pool_dst = _pad_indices(batch, _POOLPAD)

    ft = jnp.pad(features.T, ((0, _NPAD - _N), (0, 0)))
    u = jnp.stack([ft[:, :_HD], ft[:, _HD:]])

    for w_raw, om in ((W1, Om1), (W2, Om2), (W3, Om3)):
        wpt = _project_l1_rows(w_raw, _KAPPA).T
        t1 = _mm_pre(u, om.T)
        bmat = _spmm_edges(srcp, dstp, t1)
        x = _relu_half(bmat)
        for _ in range(_NITER - 1):
            g = _spmm_edges(srcp, dstp, x)
            x = _iter_step(g, wpt, bmat)
        u = x

    q = _spmm_pool(pool_src, pool_dst, u)
    return _head(q[:, :_NGRAPH, :], V0_w.T, V0_b.reshape(1, -1),
                 V1_w.T, V1_b.reshape(1, -1))
